# baseline (speedup 1.0000x reference)
import jax
import jax.numpy as jnp
from jax.experimental import pallas as pl
from jax.experimental.pallas import tpu as pltpu

_CHUNK = 256


def _split(v):
    hi = v.astype(jnp.bfloat16)
    lo = (v - hi.astype(jnp.float32)).astype(jnp.bfloat16)
    return hi, lo


def _dot(a, b):
    return jax.lax.dot_general(
        a, b, (((1,), (0,)), ((), ())), preferred_element_type=jnp.float32
    )


def _net_kernel(x_ref, adj_hbm, W1a_ref, b1a_ref, W1b_ref, b1b_ref,
                W2a_ref, b2a_ref, W2b_ref, b2b_ref,
                W3a_ref, b3a_ref, W3b_ref, b3b_ref, out_ref,
                A_f32, A_bf, sem):
    N = adj_hbm.shape[0]
    F = x_ref.shape[1]
    n_chunks = N // _CHUNK

    def chunk_copy(k):
        sl = pl.ds(k * _CHUNK, _CHUNK)
        return pltpu.make_async_copy(adj_hbm.at[sl, :], A_f32.at[sl, :], sem.at[k])

    for k in range(n_chunks):
        chunk_copy(k).start()

    g = x_ref[...].T
    hi, lo = _split(g)
    s = jnp.concatenate([hi, lo], axis=0)

    acc2 = jnp.zeros((2 * F, N), jnp.float32)
    for k in range(n_chunks):
        chunk_copy(k).wait()
        sl = slice(k * _CHUNK, (k + 1) * _CHUNK)
        a_k = A_f32[sl, :].astype(jnp.bfloat16)
        A_bf[sl, :] = a_k
        acc2 = acc2 + _dot(s[:, sl], a_k)
    acc = acc2[:F] + acc2[F:]

    A = A_bf[...]

    def mlp_fast(z, Wa_ref, ba_ref, Wb_ref, bb_ref):
        u = jnp.maximum(
            _dot(Wa_ref[...].T.astype(jnp.bfloat16), z.astype(jnp.bfloat16))
            + ba_ref[...].reshape(-1, 1), 0.0)
        return (_dot(Wb_ref[...].T.astype(jnp.bfloat16), u.astype(jnp.bfloat16))
                + bb_ref[...].reshape(-1, 1))

    def agg_fast(t):
        return t + _dot(t.astype(jnp.bfloat16), A)

    g1 = jnp.maximum(mlp_fast(g + acc, W1a_ref, b1a_ref, W1b_ref, b1b_ref), 0.0)
    g2 = jnp.maximum(mlp_fast(agg_fast(g1), W2a_ref, b2a_ref, W2b_ref, b2b_ref), 0.0)
    u3 = jnp.maximum(
        _dot(W3a_ref[...].T.astype(jnp.bfloat16),
             agg_fast(g2).astype(jnp.bfloat16))
        + b3a_ref[...].reshape(-1, 1), 0.0)
    u3_mean = jnp.mean(u3, axis=1, keepdims=True)
    m_hi, m_lo = _split(u3_mean)
    w3b = W3b_ref[...].T.astype(jnp.bfloat16)
    out = _dot(w3b, m_hi) + _dot(w3b, m_lo) + b3b_ref[...].reshape(-1, 1)
    out_ref[...] = out.T


@jax.jit
def kernel(x, adj, W1a, b1a, W1b, b1b, W2a, b2a, W2b, b2b, W3a, b3a, W3b, b3b):
    N = adj.shape[0]
    O = W3b.shape[1]
    vmem = pl.BlockSpec(memory_space=pltpu.MemorySpace.VMEM)
    return pl.pallas_call(
        _net_kernel,
        out_shape=jax.ShapeDtypeStruct((1, O), jnp.float32),
        in_specs=[vmem, pl.BlockSpec(memory_space=pltpu.MemorySpace.HBM)]
        + [vmem] * 12,
        scratch_shapes=[
            pltpu.VMEM((N, N), jnp.float32),
            pltpu.VMEM((N, N), jnp.bfloat16),
            pltpu.SemaphoreType.DMA((N // _CHUNK,)),
        ],
        compiler_params=pltpu.CompilerParams(
            vmem_limit_bytes=100 * 1024 * 1024,
        ),
    )(x, adj, W1a, b1a, W1b, b1b, W2a, b2a, W2b, b2b, W3a, b3a, W3b, b3b)


# speedup vs baseline: 1.0821x; 1.0821x over previous
import jax
import jax.numpy as jnp
from jax.experimental import pallas as pl
from jax.experimental.pallas import tpu as pltpu

_CHUNK = 256


def _split(v):
    hi = v.astype(jnp.bfloat16)
    lo = (v - hi.astype(jnp.float32)).astype(jnp.bfloat16)
    return hi, lo


def _dot(a, b):
    return jax.lax.dot_general(
        a, b, (((1,), (0,)), ((), ())), preferred_element_type=jnp.float32
    )


def _net_kernel(x_ref, adj_hbm, W1a_ref, b1a_ref, W1b_ref, b1b_ref,
                W2a_ref, b2a_ref, W2b_ref, b2b_ref,
                W3a_ref, b3a_ref, W3b_ref, b3b_ref, out_ref,
                A_f32, A_f8, sem):
    N = adj_hbm.shape[0]
    F = x_ref.shape[1]
    n_chunks = N // _CHUNK

    def chunk_copy(k):
        sl = pl.ds(k * _CHUNK, _CHUNK)
        return pltpu.make_async_copy(adj_hbm.at[sl, :], A_f32.at[sl, :], sem.at[k])

    for k in range(n_chunks):
        chunk_copy(k).start()

    g = x_ref[...].T
    hi, lo = _split(g)
    s = jnp.concatenate([hi, lo], axis=0)

    acc2 = jnp.zeros((2 * F, N), jnp.float32)
    for k in range(n_chunks):
        chunk_copy(k).wait()
        sl = slice(k * _CHUNK, (k + 1) * _CHUNK)
        a_k = A_f32[sl, :].astype(jnp.bfloat16)
        A_f8[sl, :] = A_f32[sl, :].astype(jnp.float8_e4m3fn)
        acc2 = acc2 + _dot(s[:, sl], a_k)
    acc = acc2[:F] + acc2[F:]

    A8 = A_f8[...]

    def mlp_fast(z, Wa_ref, ba_ref, Wb_ref, bb_ref):
        u = jnp.maximum(
            _dot(Wa_ref[...].T.astype(jnp.bfloat16), z.astype(jnp.bfloat16))
            + ba_ref[...].reshape(-1, 1), 0.0)
        return (_dot(Wb_ref[...].T.astype(jnp.bfloat16), u.astype(jnp.bfloat16))
                + bb_ref[...].reshape(-1, 1))

    _SCALE = 0.0078125

    def agg_fast(t):
        t8 = (t * _SCALE).astype(jnp.float8_e4m3fn)
        return t + _dot(t8, A8) * 128.0

    g1 = jnp.maximum(mlp_fast(g + acc, W1a_ref, b1a_ref, W1b_ref, b1b_ref), 0.0)
    g2 = jnp.maximum(mlp_fast(agg_fast(g1), W2a_ref, b2a_ref, W2b_ref, b2b_ref), 0.0)
    u3 = jnp.maximum(
        _dot(W3a_ref[...].T.astype(jnp.bfloat16),
             agg_fast(g2).astype(jnp.bfloat16))
        + b3a_ref[...].reshape(-1, 1), 0.0)
    u3_mean = jnp.mean(u3, axis=1, keepdims=True)
    m_hi, m_lo = _split(u3_mean)
    w3b = W3b_ref[...].T.astype(jnp.bfloat16)
    out = _dot(w3b, m_hi) + _dot(w3b, m_lo) + b3b_ref[...].reshape(-1, 1)
    out_ref[...] = out.T


@jax.jit
def kernel(x, adj, W1a, b1a, W1b, b1b, W2a, b2a, W2b, b2b, W3a, b3a, W3b, b3b):
    N = adj.shape[0]
    O = W3b.shape[1]
    vmem = pl.BlockSpec(memory_space=pltpu.MemorySpace.VMEM)
    return pl.pallas_call(
        _net_kernel,
        out_shape=jax.ShapeDtypeStruct((1, O), jnp.float32),
        in_specs=[vmem, pl.BlockSpec(memory_space=pltpu.MemorySpace.HBM)]
        + [vmem] * 12,
        scratch_shapes=[
            pltpu.VMEM((N, N), jnp.float32),
            pltpu.VMEM((N, N), jnp.float8_e4m3fn),
            pltpu.SemaphoreType.DMA((N // _CHUNK,)),
        ],
        compiler_params=pltpu.CompilerParams(
            vmem_limit_bytes=100 * 1024 * 1024,
        ),
    )(x, adj, W1a, b1a, W1b, b1b, W2a, b2a, W2b, b2b, W3a, b3a, W3b, b3b)
